# Initial kernel scaffold; baseline (speedup 1.0000x reference)
import functools

import jax
import jax.numpy as jnp
from jax import lax
from jax.experimental import pallas as pl
from jax.experimental.pallas import tpu as pltpu
from jax.experimental.pallas import tpu_sc as plsc

N = 10000
D = 128
HIDDEN = 256
E = 320000

NC = 2
NS = 16
NW = NC * NS
EPW = E // NW
CHUNK = 80
NCHUNK = EPW // CHUNK
ROWS_PT = N // NS
RCHUNK = 125
NRCHUNK = ROWS_PT // RCHUNK

_mesh = plsc.VectorSubcoreMesh(core_axis_name="c", subcore_axis_name="s")


@functools.partial(
    pl.kernel,
    out_type=jax.ShapeDtypeStruct((NC, N), jnp.float32),
    mesh=_mesh,
    scratch_types=[
        pltpu.VMEM((NCHUNK, CHUNK), jnp.int32),
        pltpu.VMEM((CHUNK,), jnp.float32),
        pltpu.VMEM((N,), jnp.float32),
        pltpu.VMEM_SHARED((N,), jnp.float32),
        pltpu.SemaphoreType.DMA,
    ],
)
def _deg_hist(dst_hbm, degp_hbm, idx_v, ones_v, stage_v, hist_sh, sem):
    c = lax.axis_index("c")
    s = lax.axis_index("s")
    w = c * NS + s

    @pl.loop(0, CHUNK, step=16)
    def _(i):
        ones_v[pl.ds(i, 16)] = jnp.ones((16,), jnp.float32)

    @pl.when(s == 0)
    def _():
        @pl.loop(0, N, step=16)
        def _(i):
            stage_v[pl.ds(i, 16)] = jnp.zeros((16,), jnp.float32)

        pltpu.sync_copy(stage_v, hist_sh)

    plsc.subcore_barrier()

    pltpu.async_copy(dst_hbm.at[w], idx_v, sem).wait()

    @pl.loop(0, NCHUNK)
    def _(j):
        pltpu.sync_copy(ones_v, hist_sh.at[idx_v.at[j]], add=True)

    plsc.subcore_barrier()

    @pl.when(s == 0)
    def _():
        pltpu.sync_copy(hist_sh, stage_v)
        pltpu.sync_copy(stage_v, degp_hbm.at[c])


@functools.partial(
    pl.kernel,
    out_type=jax.ShapeDtypeStruct((NC, N, D), jnp.float32),
    mesh=_mesh,
    scratch_types=[
        pltpu.VMEM((NCHUNK, CHUNK), jnp.int32),
        pltpu.VMEM((NCHUNK, CHUNK), jnp.int32),
        pltpu.VMEM((CHUNK, D), jnp.float32),
        pltpu.VMEM((RCHUNK, D), jnp.float32),
        pltpu.VMEM_SHARED((N, D), jnp.float32),
        pltpu.SemaphoreType.DMA,
        pltpu.SemaphoreType.DMA,
    ],
)
def _edge_scatter(y_hbm, src_hbm, dst_hbm, accp_hbm,
                  src_v, dst_v, rows_v, stage_v, acc_sh, sem0, sem1):
    c = lax.axis_index("c")
    s = lax.axis_index("s")
    w = c * NS + s

    pltpu.async_copy(src_hbm.at[w], src_v, sem0)
    pltpu.async_copy(dst_hbm.at[w], dst_v, sem1)

    stage_flat = stage_v.reshape((RCHUNK * D,))

    @pl.loop(0, RCHUNK * D, step=16)
    def _(i):
        stage_flat[pl.ds(i, 16)] = jnp.zeros((16,), jnp.float32)

    @pl.loop(0, NRCHUNK)
    def _(k):
        pltpu.sync_copy(stage_v, acc_sh.at[pl.ds(s * ROWS_PT + k * RCHUNK, RCHUNK)])

    pltpu.make_async_copy(src_hbm.at[w], src_v, sem0).wait()
    pltpu.make_async_copy(dst_hbm.at[w], dst_v, sem1).wait()

    plsc.subcore_barrier()

    @pl.loop(0, NCHUNK)
    def _(j):
        pltpu.sync_copy(y_hbm.at[src_v.at[j]], rows_v)
        pltpu.sync_copy(rows_v, acc_sh.at[dst_v.at[j]], add=True)

    plsc.subcore_barrier()

    @pl.loop(0, NRCHUNK)
    def _(k):
        base = s * ROWS_PT + k * RCHUNK
        pltpu.sync_copy(acc_sh.at[pl.ds(base, RCHUNK)], stage_v)
        pltpu.sync_copy(stage_v, accp_hbm.at[c].at[pl.ds(base, RCHUNK)])


def _mlp_body(x_ref, w1t_ref, b1_ref, w2t_ref, degp_ref, y_ref):
    h = jnp.dot(x_ref[...], w1t_ref[...], preferred_element_type=jnp.float32)
    h = jnp.maximum(h + b1_ref[...], 0.0)
    x2 = jnp.dot(h, w2t_ref[...], preferred_element_type=jnp.float32)
    dinv = lax.rsqrt(degp_ref[0] + degp_ref[1] + 1.0)
    y_ref[...] = x2 * dinv


def _mlp(x, w1t, b1, w2t, degp):
    return pl.pallas_call(
        _mlp_body,
        out_shape=jax.ShapeDtypeStruct((N, D), jnp.float32),
    )(x, w1t, b1, w2t, degp)


def _finish_body(accp_ref, y_ref, degp_ref, b2_ref, out_ref):
    dinv = lax.rsqrt(degp_ref[0] + degp_ref[1] + 1.0)
    out_ref[...] = dinv * (accp_ref[0] + accp_ref[1] + y_ref[...]) + b2_ref[...]


def _finish(accp, y, degp, b2):
    return pl.pallas_call(
        _finish_body,
        out_shape=jax.ShapeDtypeStruct((N, D), jnp.float32),
    )(accp, y, degp, b2)


def kernel(node_features, edge_index, W1, b1, W2, b2):
    src = edge_index[0].astype(jnp.int32).reshape(NW, NCHUNK, CHUNK)
    dst = edge_index[1].astype(jnp.int32).reshape(NW, NCHUNK, CHUNK)

    degp = _deg_hist(dst)
    degp3 = degp.reshape(NC, N, 1)

    y = _mlp(node_features, W1.T, b1.reshape(1, HIDDEN), W2.T, degp3)

    accp = _edge_scatter(y, src, dst)

    return _finish(accp, y, degp3, b2.reshape(1, D))


# speedup vs baseline: 27.4866x; 27.4866x over previous
import functools

import jax
import jax.numpy as jnp
from jax import lax
from jax.experimental import pallas as pl
from jax.experimental.pallas import tpu as pltpu
from jax.experimental.pallas import tpu_sc as plsc

N = 10000
D = 128
HIDDEN = 256
E = 320000

NC = 2
NS = 16
NW = NC * NS
EPW = E // NW
CHUNK = 80
NCHUNK = EPW // CHUNK
N_PAD = 10240
ROWS_PT = N_PAD // NS
NRCHUNK = ROWS_PT // CHUNK

_mesh = plsc.VectorSubcoreMesh(core_axis_name="c", subcore_axis_name="s")


@functools.partial(
    pl.kernel,
    out_type=jax.ShapeDtypeStruct((NC, N), jnp.float32),
    mesh=_mesh,
    scratch_types=[
        pltpu.VMEM((NCHUNK, CHUNK), jnp.int32),
        pltpu.VMEM((CHUNK,), jnp.float32),
        pltpu.VMEM((N,), jnp.float32),
        pltpu.VMEM_SHARED((N,), jnp.float32),
        pltpu.SemaphoreType.DMA,
    ],
)
def _deg_hist(dst_hbm, degp_hbm, idx_v, ones_v, stage_v, hist_sh, sem):
    c = lax.axis_index("c")
    s = lax.axis_index("s")
    w = c * NS + s

    @pl.loop(0, CHUNK, step=16)
    def _(i):
        ones_v[pl.ds(i, 16)] = jnp.ones((16,), jnp.float32)

    @pl.when(s == 0)
    def _():
        @pl.loop(0, N, step=16)
        def _(i):
            stage_v[pl.ds(i, 16)] = jnp.zeros((16,), jnp.float32)

        pltpu.sync_copy(stage_v, hist_sh)

    plsc.subcore_barrier()

    pltpu.async_copy(dst_hbm.at[w], idx_v, sem).wait()

    @pl.loop(0, NCHUNK)
    def _(j):
        pltpu.sync_copy(ones_v, hist_sh.at[idx_v.at[j]], add=True)

    plsc.subcore_barrier()

    @pl.when(s == 0)
    def _():
        pltpu.sync_copy(hist_sh, stage_v)
        pltpu.sync_copy(stage_v, degp_hbm.at[c])


@functools.partial(
    pl.kernel,
    out_type=jax.ShapeDtypeStruct((NC, N_PAD, D), jnp.float32),
    mesh=_mesh,
    scratch_types=[
        pltpu.VMEM((NCHUNK, CHUNK), jnp.int32),
        pltpu.VMEM((NCHUNK, CHUNK), jnp.int32),
        pltpu.VMEM((CHUNK, D), jnp.float32),
        pltpu.VMEM_SHARED((N_PAD, D), jnp.float32),
        pltpu.SemaphoreType.DMA,
        pltpu.SemaphoreType.DMA,
    ],
)
def _edge_scatter(y_hbm, src_hbm, dst_hbm, accp_hbm,
                  src_v, dst_v, rows_v, acc_sh, sem0, sem1):
    c = lax.axis_index("c")
    s = lax.axis_index("s")
    w = c * NS + s

    pltpu.async_copy(src_hbm.at[w], src_v, sem0)
    pltpu.async_copy(dst_hbm.at[w], dst_v, sem1)

    @pl.loop(0, CHUNK)
    def _(r):
        @pl.loop(0, D, step=16)
        def _(i):
            rows_v[r, pl.ds(i, 16)] = jnp.zeros((16,), jnp.float32)

    @pl.loop(0, NRCHUNK)
    def _(k):
        pltpu.sync_copy(rows_v, acc_sh.at[pl.ds(s * ROWS_PT + k * CHUNK, CHUNK)])

    pltpu.make_async_copy(src_hbm.at[w], src_v, sem0).wait()
    pltpu.make_async_copy(dst_hbm.at[w], dst_v, sem1).wait()

    plsc.subcore_barrier()

    @pl.loop(0, NCHUNK)
    def _(j):
        pltpu.sync_copy(y_hbm.at[src_v.at[j]], rows_v)
        pltpu.sync_copy(rows_v, acc_sh.at[dst_v.at[j]], add=True)

    plsc.subcore_barrier()

    @pl.loop(0, NRCHUNK)
    def _(k):
        base = s * ROWS_PT + k * CHUNK
        pltpu.sync_copy(acc_sh.at[pl.ds(base, CHUNK)], rows_v)
        pltpu.sync_copy(rows_v, accp_hbm.at[c].at[pl.ds(base, CHUNK)])


def _mlp_body(x_ref, w1t_ref, b1_ref, w2t_ref, degp_ref, y_ref):
    h = jnp.dot(x_ref[...], w1t_ref[...], preferred_element_type=jnp.float32)
    h = jnp.maximum(h + b1_ref[...], 0.0)
    x2 = jnp.dot(h, w2t_ref[...], preferred_element_type=jnp.float32)
    dinv = lax.rsqrt(degp_ref[0] + degp_ref[1] + 1.0)
    y_ref[...] = x2 * dinv


def _mlp(x, w1t, b1, w2t, degp):
    return pl.pallas_call(
        _mlp_body,
        out_shape=jax.ShapeDtypeStruct((N, D), jnp.float32),
    )(x, w1t, b1, w2t, degp)


def _finish_body(accp_ref, y_ref, degp_ref, b2_ref, out_ref):
    dinv = lax.rsqrt(degp_ref[0] + degp_ref[1] + 1.0)
    acc = accp_ref[0, 0:N, :] + accp_ref[1, 0:N, :]
    out_ref[...] = dinv * (acc + y_ref[...]) + b2_ref[...]


def _finish(accp, y, degp, b2):
    return pl.pallas_call(
        _finish_body,
        out_shape=jax.ShapeDtypeStruct((N, D), jnp.float32),
    )(accp, y, degp, b2)


def kernel(node_features, edge_index, W1, b1, W2, b2):
    src = edge_index[0].astype(jnp.int32).reshape(NW, NCHUNK, CHUNK)
    dst = edge_index[1].astype(jnp.int32).reshape(NW, NCHUNK, CHUNK)

    degp = _deg_hist(dst)
    degp3 = degp.reshape(NC, N, 1)

    y = _mlp(node_features, W1.T, b1.reshape(1, HIDDEN), W2.T, degp3)

    accp = _edge_scatter(y, src, dst)

    return _finish(accp, y, degp3, b2.reshape(1, D))
